# Initial kernel scaffold; baseline (speedup 1.0000x reference)
import jax
import jax.numpy as jnp
from jax.experimental import pallas as pl


def kernel(x, LUT):
    raise NotImplementedError("write your pallas kernel here")


# speedup vs baseline: 158.7318x; 158.7318x over previous
import functools
import jax
import jax.numpy as jnp
import numpy as np
from jax import lax
from jax.experimental import pallas as pl
from jax.experimental.pallas import tpu as pltpu
from jax.experimental.pallas import tpu_sc as plsc

DIM = 33
LUT_SIZE = 3 * DIM * DIM * DIM
NC, NS, L = 2, 16, 16
NW = NC * NS
INV_BINSIZE = np.float32((DIM - 1) / 1.000001)


def _lut3d_sc(x_flat, lut_flat, B, H, W):
    N = B * H * W
    per_w = N // NW
    P = 2048
    n_blk = per_w // P
    plane = H * W

    mesh = plsc.VectorSubcoreMesh(
        core_axis_name="c", subcore_axis_name="s",
        num_cores=NC, num_subcores=NS)

    @functools.partial(
        pl.kernel,
        out_type=jax.ShapeDtypeStruct((B * 3 * H * W,), jnp.float32),
        mesh=mesh,
        compiler_params=pltpu.CompilerParams(needs_layout_passes=False),
        scratch_types=[
            pltpu.VMEM((LUT_SIZE,), jnp.float32),
            pltpu.VMEM((P,), jnp.float32), pltpu.VMEM((P,), jnp.float32),
            pltpu.VMEM((P,), jnp.float32), pltpu.VMEM((P,), jnp.float32),
            pltpu.VMEM((P,), jnp.float32), pltpu.VMEM((P,), jnp.float32),
        ],
    )
    def lut_kernel(x_hbm, lut_hbm, out_hbm,
                   lut_v, in_r, in_g, in_b, out_r, out_g, out_b):
        wid = lax.axis_index("s") * NC + lax.axis_index("c")
        ppi = plane
        img = (wid * per_w) // ppi
        inner = (wid * per_w) % ppi
        base_r = (img * 3 + 0) * plane + inner
        base_g = (img * 3 + 1) * plane + inner
        base_b = (img * 3 + 2) * plane + inner

        pltpu.sync_copy(lut_hbm, lut_v)

        def blk_body(blk, _):
            off = blk * P
            pltpu.sync_copy(x_hbm.at[pl.ds(base_r + off, P)], in_r)
            pltpu.sync_copy(x_hbm.at[pl.ds(base_g + off, P)], in_g)
            pltpu.sync_copy(x_hbm.at[pl.ds(base_b + off, P)], in_b)

            def vec_body(i, _):
                s = i * L
                r = in_r[pl.ds(s, L)]
                g = in_g[pl.ds(s, L)]
                b = in_b[pl.ds(s, L)]
                rf = r * INV_BINSIZE
                gf = g * INV_BINSIZE
                bf = b * INV_BINSIZE
                rid = jnp.clip(rf.astype(jnp.int32), 0, DIM - 2)
                gid = jnp.clip(gf.astype(jnp.int32), 0, DIM - 2)
                bid = jnp.clip(bf.astype(jnp.int32), 0, DIM - 2)
                rd = rf - rid.astype(jnp.float32)
                gd = gf - gid.astype(jnp.float32)
                bd = bf - bid.astype(jnp.float32)
                idx = rid + gid * DIM + bid * (DIM * DIM)

                def interp(c3):
                    bi = idx + c3 * (DIM * DIM * DIM)
                    v000 = plsc.load_gather(lut_v, [bi])
                    v100 = plsc.load_gather(lut_v, [bi + 1])
                    v010 = plsc.load_gather(lut_v, [bi + DIM])
                    v110 = plsc.load_gather(lut_v, [bi + (DIM + 1)])
                    v001 = plsc.load_gather(lut_v, [bi + DIM * DIM])
                    v101 = plsc.load_gather(lut_v, [bi + (DIM * DIM + 1)])
                    v011 = plsc.load_gather(lut_v, [bi + (DIM * DIM + DIM)])
                    v111 = plsc.load_gather(lut_v, [bi + (DIM * DIM + DIM + 1)])
                    a0 = v000 + rd * (v100 - v000)
                    a1 = v010 + rd * (v110 - v010)
                    a2 = v001 + rd * (v101 - v001)
                    a3 = v011 + rd * (v111 - v011)
                    c0 = a0 + gd * (a1 - a0)
                    c1 = a2 + gd * (a3 - a2)
                    return c0 + bd * (c1 - c0)

                out_r[pl.ds(s, L)] = interp(0)
                out_g[pl.ds(s, L)] = interp(1)
                out_b[pl.ds(s, L)] = interp(2)
                return 0

            lax.fori_loop(0, P // L, vec_body, 0)
            pltpu.sync_copy(out_r, out_hbm.at[pl.ds(base_r + off, P)])
            pltpu.sync_copy(out_g, out_hbm.at[pl.ds(base_g + off, P)])
            pltpu.sync_copy(out_b, out_hbm.at[pl.ds(base_b + off, P)])
            return 0

        lax.fori_loop(0, n_blk, blk_body, 0)

    return lut_kernel(x_flat, lut_flat)


def kernel(x, LUT):
    B, C, H, W = x.shape
    out_flat = _lut3d_sc(x.reshape(-1), LUT.reshape(-1), B, H, W)
    return out_flat.reshape(B, C, H, W)


# speedup vs baseline: 288.6364x; 1.8184x over previous
import functools
import jax
import jax.numpy as jnp
import numpy as np
from jax import lax
from jax.experimental import pallas as pl
from jax.experimental.pallas import tpu as pltpu
from jax.experimental.pallas import tpu_sc as plsc

DIM = 33
CSIZE = DIM * DIM * DIM
NC, NS, L = 2, 16, 16
NW = NC * NS
INV_BINSIZE = np.float32((DIM - 1) / 1.000001)
P = 1024


def _lut3d_sc(x_flat, lut0, lut1, lut2, n_pix, plane):
    per_w = n_pix // NW
    n_blk = per_w // P
    n_pair = n_blk // 2

    mesh = plsc.VectorSubcoreMesh(
        core_axis_name="c", subcore_axis_name="s",
        num_cores=NC, num_subcores=NS)

    vbuf = lambda: pltpu.VMEM((P,), jnp.float32)

    @functools.partial(
        pl.kernel,
        out_type=jax.ShapeDtypeStruct((3 * n_pix,), jnp.float32),
        mesh=mesh,
        compiler_params=pltpu.CompilerParams(needs_layout_passes=False),
        scratch_types=[
            pltpu.VMEM((CSIZE,), jnp.float32),
            pltpu.VMEM((CSIZE,), jnp.float32),
            pltpu.VMEM((CSIZE,), jnp.float32),
            vbuf(), vbuf(), vbuf(), vbuf(), vbuf(), vbuf(),
            vbuf(), vbuf(), vbuf(), vbuf(), vbuf(), vbuf(),
            pltpu.SemaphoreType.DMA, pltpu.SemaphoreType.DMA,
            pltpu.SemaphoreType.DMA, pltpu.SemaphoreType.DMA,
        ],
    )
    def lut_kernel(x_hbm, l0_hbm, l1_hbm, l2_hbm, out_hbm,
                   l0, l1, l2,
                   i0r, i0g, i0b, i1r, i1g, i1b,
                   o0r, o0g, o0b, o1r, o1g, o1b,
                   s_in0, s_in1, s_out0, s_out1):
        wid = lax.axis_index("s") * NC + lax.axis_index("c")
        img = wid // 2
        col0 = (wid % 2) * per_w
        bases = tuple((img * 3 + c) * plane + col0 for c in range(3))
        in0, in1 = (i0r, i0g, i0b), (i1r, i1g, i1b)
        out0, out1 = (o0r, o0g, o0b), (o1r, o1g, o1b)

        pltpu.sync_copy(l0_hbm, l0)
        pltpu.sync_copy(l1_hbm, l1)
        pltpu.sync_copy(l2_hbm, l2)

        def in_descs(blk, bufs, sem):
            off = blk * P
            return [pltpu.make_async_copy(
                x_hbm.at[pl.ds(bases[c] + off, P)], bufs[c], sem)
                for c in range(3)]

        def out_descs(blk, bufs, sem):
            off = blk * P
            return [pltpu.make_async_copy(
                bufs[c], out_hbm.at[pl.ds(bases[c] + off, P)], sem)
                for c in range(3)]

        def start(descs):
            for d in descs:
                d.start()

        def wait(descs):
            for d in descs:
                d.wait()

        def compute(src, dst):
            @plsc.parallel_loop(0, P // L, 1, unroll=2)
            def vec_body(i):
                s = i * L
                r = src[0][pl.ds(s, L)]
                g = src[1][pl.ds(s, L)]
                b = src[2][pl.ds(s, L)]
                rf = r * INV_BINSIZE
                gf = g * INV_BINSIZE
                bf = b * INV_BINSIZE
                rid = jnp.clip(rf.astype(jnp.int32), 0, DIM - 2)
                gid = jnp.clip(gf.astype(jnp.int32), 0, DIM - 2)
                bid = jnp.clip(bf.astype(jnp.int32), 0, DIM - 2)
                rd = rf - rid.astype(jnp.float32)
                gd = gf - gid.astype(jnp.float32)
                bd = bf - bid.astype(jnp.float32)
                tr = 1.0 - rd
                tg = 1.0 - gd
                tb = 1.0 - bd
                p00 = tg * tb
                p10 = gd * tb
                p01 = tg * bd
                p11 = gd * bd
                w000 = tr * p00
                w100 = rd * p00
                w010 = tr * p10
                w110 = rd * p10
                w001 = tr * p01
                w101 = rd * p01
                w011 = tr * p11
                w111 = rd * p11

                i000 = rid + gid * DIM + bid * (DIM * DIM)
                i100 = i000 + 1
                i010 = i000 + DIM
                i110 = i000 + (DIM + 1)
                i001 = i000 + DIM * DIM
                i101 = i000 + (DIM * DIM + 1)
                i011 = i000 + (DIM * DIM + DIM)
                i111 = i000 + (DIM * DIM + DIM + 1)

                def interp(lv):
                    return (w000 * plsc.load_gather(lv, [i000])
                            + w100 * plsc.load_gather(lv, [i100])
                            + w010 * plsc.load_gather(lv, [i010])
                            + w110 * plsc.load_gather(lv, [i110])
                            + w001 * plsc.load_gather(lv, [i001])
                            + w101 * plsc.load_gather(lv, [i101])
                            + w011 * plsc.load_gather(lv, [i011])
                            + w111 * plsc.load_gather(lv, [i111]))

                dst[0][pl.ds(s, L)] = interp(l0)
                dst[1][pl.ds(s, L)] = interp(l1)
                dst[2][pl.ds(s, L)] = interp(l2)

        start(in_descs(0, in0, s_in0))

        def pair_body(p, _):
            b0 = 2 * p
            b1 = b0 + 1
            start(in_descs(b1, in1, s_in1))
            wait(in_descs(b0, in0, s_in0))

            @pl.when(p >= 1)
            def _():
                wait(out_descs(b0 - 2, out0, s_out0))

            compute(in0, out0)
            start(out_descs(b0, out0, s_out0))

            @pl.when(p + 1 < n_pair)
            def _():
                start(in_descs(b0 + 2, in0, s_in0))

            wait(in_descs(b1, in1, s_in1))

            @pl.when(p >= 1)
            def _():
                wait(out_descs(b1 - 2, out1, s_out1))

            compute(in1, out1)
            start(out_descs(b1, out1, s_out1))
            return 0

        lax.fori_loop(0, n_pair, pair_body, 0)
        wait(out_descs(n_blk - 2, out0, s_out0))
        wait(out_descs(n_blk - 1, out1, s_out1))

    return lut_kernel(x_flat, lut0, lut1, lut2)


def kernel(x, LUT):
    B, C, H, W = x.shape
    plane = H * W
    lut_flat = LUT.reshape(3, CSIZE)
    out_flat = _lut3d_sc(x.reshape(-1),
                         lut_flat[0], lut_flat[1], lut_flat[2],
                         B * plane, plane)
    return out_flat.reshape(B, C, H, W)
